# Initial kernel scaffold; baseline (speedup 1.0000x reference)
import jax
import jax.numpy as jnp
from jax.experimental import pallas as pl


def kernel(x, edge_index, edge_weight, W, b):
    raise NotImplementedError("write your pallas kernel here")


# speedup vs baseline: 3.2416x; 3.2416x over previous
import functools

import jax
import jax.numpy as jnp
from jax import lax
from jax.experimental import pallas as pl
from jax.experimental.pallas import tpu as pltpu
from jax.experimental.pallas import tpu_sc as plsc

N_NODES = 10000
N_EDGES = 320000
D = 128

NC = 2
NS = 16
LANES = 16

CHUNK = 128
CHUNKS_PER_TILE = 80
E_PAD = NC * NS * CHUNKS_PER_TILE * CHUNK
STRIPE = 624
TAIL = N_NODES - NS * STRIPE


def _matmul(x, W):
    def body(x_ref, w_ref, o_ref):
        o_ref[...] = jnp.dot(x_ref[...], w_ref[...],
                             preferred_element_type=jnp.float32)

    blk = 1000
    return pl.pallas_call(
        body,
        grid=(N_NODES // blk,),
        in_specs=[
            pl.BlockSpec((blk, D), lambda i: (i, 0)),
            pl.BlockSpec((D, D), lambda i: (0, 0)),
        ],
        out_specs=pl.BlockSpec((blk, D), lambda i: (i, 0)),
        out_shape=jax.ShapeDtypeStruct((N_NODES, D), jnp.float32),
    )(x, W)


def _combine(partials, b2):
    def body(p_ref, b_ref, o_ref):
        o_ref[...] = p_ref[0] + p_ref[1] + b_ref[...]

    blk = 1000
    return pl.pallas_call(
        body,
        grid=(N_NODES // blk,),
        in_specs=[
            pl.BlockSpec((2, blk, D), lambda i: (0, i, 0)),
            pl.BlockSpec((1, D), lambda i: (0, 0)),
        ],
        out_specs=pl.BlockSpec((blk, D), lambda i: (i, 0)),
        out_shape=jax.ShapeDtypeStruct((N_NODES, D), jnp.float32),
    )(partials, b2)


def _sc_spmm(src3, dst3, w3, support):
    mesh = plsc.VectorSubcoreMesh(core_axis_name="c", subcore_axis_name="s")

    @functools.partial(
        pl.kernel,
        out_type=jax.ShapeDtypeStruct((NC, N_NODES, D), jnp.float32),
        mesh=mesh,
        scratch_types=[
            pltpu.VMEM((CHUNKS_PER_TILE, CHUNK), jnp.int32),
            pltpu.VMEM((CHUNKS_PER_TILE, CHUNK), jnp.int32),
            pltpu.VMEM((CHUNKS_PER_TILE, CHUNK), jnp.float32),
            pltpu.VMEM((CHUNK, D), jnp.float32),
            pltpu.VMEM_SHARED((N_NODES, D), jnp.float32),
            pltpu.SemaphoreType.DMA,
        ],
    )
    def k(src_hbm, dst_hbm, w_hbm, sup_hbm, out_hbm,
          src_v, dst_v, w_v, rows, acc, sem):
        c = lax.axis_index("c")
        s = lax.axis_index("s")
        wid = c * NS + s

        def zbody(i, carry):
            for dd in range(D // LANES):
                rows[i, pl.ds(dd * LANES, LANES)] = jnp.zeros(
                    (LANES,), jnp.float32)
            return carry

        lax.fori_loop(0, CHUNK, zbody, 0)
        for z in range(4):
            pltpu.sync_copy(rows.at[pl.ds(0, CHUNK)],
                            acc.at[pl.ds(s * STRIPE + z * CHUNK, CHUNK)])
        pltpu.sync_copy(rows.at[pl.ds(0, STRIPE - 4 * CHUNK)],
                        acc.at[pl.ds(s * STRIPE + 4 * CHUNK,
                                     STRIPE - 4 * CHUNK)])

        @pl.when(s == NS - 1)
        def _():
            pltpu.sync_copy(rows.at[pl.ds(0, TAIL)],
                            acc.at[pl.ds(NS * STRIPE, TAIL)])

        plsc.subcore_barrier()

        base = wid * CHUNKS_PER_TILE
        pltpu.sync_copy(src_hbm.at[pl.ds(base, CHUNKS_PER_TILE)], src_v)
        pltpu.sync_copy(dst_hbm.at[pl.ds(base, CHUNKS_PER_TILE)], dst_v)
        pltpu.sync_copy(w_hbm.at[pl.ds(base, CHUNKS_PER_TILE)], w_v)

        def chunk_body(j, carry):
            pltpu.async_copy(sup_hbm.at[src_v.at[j]], rows, sem).wait()

            def mul_body(g, inner):
                wv = w_v[j, pl.ds(g * LANES, LANES)]
                for e in range(LANES):
                    i = g * LANES + e
                    wsc = wv[e]
                    for dd in range(D // LANES):
                        sl = pl.ds(dd * LANES, LANES)
                        rows[i, sl] = rows[i, sl] * wsc
                return inner

            lax.fori_loop(0, CHUNK // LANES, mul_body, 0)

            pltpu.sync_copy(rows, acc.at[dst_v.at[j]], add=True)
            return carry

        lax.fori_loop(0, CHUNKS_PER_TILE, chunk_body, 0)
        plsc.subcore_barrier()

        pltpu.sync_copy(acc.at[pl.ds(s * STRIPE, STRIPE)],
                        out_hbm.at[c, pl.ds(s * STRIPE, STRIPE)])

        @pl.when(s == NS - 1)
        def _():
            pltpu.sync_copy(acc.at[pl.ds(NS * STRIPE, TAIL)],
                            out_hbm.at[c, pl.ds(NS * STRIPE, TAIL)])

    return k(src3, dst3, w3, support)


def kernel(x, edge_index, edge_weight, W, b):
    support = _matmul(x, W)

    src = edge_index[1].astype(jnp.int32)
    dst = edge_index[0].astype(jnp.int32)
    pad = E_PAD - N_EDGES
    zi = jnp.zeros((pad,), jnp.int32)
    src3 = jnp.concatenate([src, zi]).reshape(E_PAD // CHUNK, CHUNK)
    dst3 = jnp.concatenate([dst, zi]).reshape(E_PAD // CHUNK, CHUNK)
    w3 = jnp.concatenate(
        [edge_weight.astype(jnp.float32), jnp.zeros((pad,), jnp.float32)]
    ).reshape(E_PAD // CHUNK, CHUNK)

    partials = _sc_spmm(src3, dst3, w3, support)
    return _combine(partials, b.reshape(1, D))


# speedup vs baseline: 3.7761x; 1.1649x over previous
import functools

import jax
import jax.numpy as jnp
from jax import lax
from jax.experimental import pallas as pl
from jax.experimental.pallas import tpu as pltpu
from jax.experimental.pallas import tpu_sc as plsc

N_NODES = 10000
N_EDGES = 320000
D = 128

NC = 2
NS = 16
LANES = 16

CHUNK = 128
CHUNKS_PER_TILE = 80
GRP = 8
E_PAD = NC * NS * CHUNKS_PER_TILE * CHUNK
STRIPE = 624
TAIL = N_NODES - NS * STRIPE


def _matmul(x, W):
    def body(x_ref, w_ref, o_ref):
        o_ref[...] = jnp.dot(x_ref[...], w_ref[...],
                             preferred_element_type=jnp.float32)

    blk = 1000
    return pl.pallas_call(
        body,
        grid=(N_NODES // blk,),
        in_specs=[
            pl.BlockSpec((blk, D), lambda i: (i, 0)),
            pl.BlockSpec((D, D), lambda i: (0, 0)),
        ],
        out_specs=pl.BlockSpec((blk, D), lambda i: (i, 0)),
        out_shape=jax.ShapeDtypeStruct((N_NODES, D), jnp.float32),
    )(x, W)


def _combine(partials, b2):
    def body(p_ref, b_ref, o_ref):
        o_ref[...] = p_ref[0] + p_ref[1] + b_ref[...]

    blk = 1000
    return pl.pallas_call(
        body,
        grid=(N_NODES // blk,),
        in_specs=[
            pl.BlockSpec((2, blk, D), lambda i: (0, i, 0)),
            pl.BlockSpec((1, D), lambda i: (0, 0)),
        ],
        out_specs=pl.BlockSpec((blk, D), lambda i: (i, 0)),
        out_shape=jax.ShapeDtypeStruct((N_NODES, D), jnp.float32),
    )(partials, b2)


def _sc_spmm(src3, dst3, w3, support):
    mesh = plsc.VectorSubcoreMesh(core_axis_name="c", subcore_axis_name="s")

    @functools.partial(
        pl.kernel,
        out_type=jax.ShapeDtypeStruct((NC, N_NODES, D), jnp.float32),
        mesh=mesh,
        scratch_types=[
            pltpu.VMEM((GRP, CHUNK), jnp.int32),
            pltpu.VMEM((GRP, CHUNK), jnp.int32),
            pltpu.VMEM((GRP, CHUNK), jnp.float32),
            pltpu.VMEM((CHUNK, D), jnp.float32),
            pltpu.VMEM((CHUNK, D), jnp.float32),
            pltpu.VMEM_SHARED((N_NODES, D), jnp.float32),
            pltpu.SemaphoreType.DMA,
            pltpu.SemaphoreType.DMA,
        ],
    )
    def k(src_hbm, dst_hbm, w_hbm, sup_hbm, out_hbm,
          src_v, dst_v, w_v, rows_a, rows_b, acc, sem_a, sem_b):
        c = lax.axis_index("c")
        s = lax.axis_index("s")
        wid = c * NS + s
        base = wid * CHUNKS_PER_TILE

        def zbody(i, carry):
            for dd in range(D // LANES):
                rows_b[i, pl.ds(dd * LANES, LANES)] = jnp.zeros(
                    (LANES,), jnp.float32)
            return carry

        lax.fori_loop(0, CHUNK, zbody, 0)

        for z in range(4):
            pltpu.sync_copy(rows_b.at[pl.ds(0, CHUNK)],
                            acc.at[pl.ds(s * STRIPE + z * CHUNK, CHUNK)])
        pltpu.sync_copy(rows_b.at[pl.ds(0, STRIPE - 4 * CHUNK)],
                        acc.at[pl.ds(s * STRIPE + 4 * CHUNK,
                                     STRIPE - 4 * CHUNK)])

        @pl.when(s == NS - 1)
        def _():
            pltpu.sync_copy(rows_b.at[pl.ds(0, TAIL)],
                            acc.at[pl.ds(NS * STRIPE, TAIL)])

        plsc.subcore_barrier()

        def scale(buf, j):
            def mul_body(g, inner):
                i = g * LANES
                wv = w_v[j, pl.ds(i, LANES)]
                for e in range(LANES):
                    wsc = wv[e]
                    for dd in range(D // LANES):
                        sl = pl.ds(dd * LANES, LANES)
                        buf[i + e, sl] = buf[i + e, sl] * wsc
                return inner

            lax.fori_loop(0, CHUNK // LANES, mul_body, 0)

        def group_body(g, carry):
            gbase = base + g * GRP
            pltpu.sync_copy(src_hbm.at[pl.ds(gbase, GRP)], src_v)
            pltpu.sync_copy(dst_hbm.at[pl.ds(gbase, GRP)], dst_v)
            pltpu.sync_copy(w_hbm.at[pl.ds(gbase, GRP)], w_v)
            pltpu.async_copy(sup_hbm.at[src_v.at[0]], rows_a, sem_a)

            def pair_body(t, inner):
                j0 = 2 * t
                j1 = 2 * t + 1
                pltpu.async_copy(sup_hbm.at[src_v.at[j1]], rows_b, sem_b)
                pltpu.make_async_copy(sup_hbm.at[src_v.at[j0]], rows_a,
                                      sem_a).wait()
                scale(rows_a, j0)
                pltpu.sync_copy(rows_a, acc.at[dst_v.at[j0]], add=True)

                @pl.when(t < GRP // 2 - 1)
                def _():
                    pltpu.async_copy(sup_hbm.at[src_v.at[j0 + 2]], rows_a,
                                     sem_a)

                pltpu.make_async_copy(sup_hbm.at[src_v.at[j1]], rows_b,
                                      sem_b).wait()
                scale(rows_b, j1)
                pltpu.sync_copy(rows_b, acc.at[dst_v.at[j1]], add=True)
                return inner

            lax.fori_loop(0, GRP // 2, pair_body, 0)
            return carry

        lax.fori_loop(0, CHUNKS_PER_TILE // GRP, group_body, 0)
        plsc.subcore_barrier()

        pltpu.sync_copy(acc.at[pl.ds(s * STRIPE, STRIPE)],
                        out_hbm.at[c, pl.ds(s * STRIPE, STRIPE)])

        @pl.when(s == NS - 1)
        def _():
            pltpu.sync_copy(acc.at[pl.ds(NS * STRIPE, TAIL)],
                            out_hbm.at[c, pl.ds(NS * STRIPE, TAIL)])

    return k(src3, dst3, w3, support)


def kernel(x, edge_index, edge_weight, W, b):
    support = _matmul(x, W)

    src = edge_index[1].astype(jnp.int32)
    dst = edge_index[0].astype(jnp.int32)
    pad = E_PAD - N_EDGES
    zi = jnp.zeros((pad,), jnp.int32)
    src3 = jnp.concatenate([src, zi]).reshape(E_PAD // CHUNK, CHUNK)
    dst3 = jnp.concatenate([dst, zi]).reshape(E_PAD // CHUNK, CHUNK)
    w3 = jnp.concatenate(
        [edge_weight.astype(jnp.float32), jnp.zeros((pad,), jnp.float32)]
    ).reshape(E_PAD // CHUNK, CHUNK)

    partials = _sc_spmm(src3, dst3, w3, support)
    return _combine(partials, b.reshape(1, D))


# speedup vs baseline: 9.4539x; 2.5036x over previous
import functools

import jax
import jax.numpy as jnp
from jax import lax
from jax.experimental import pallas as pl
from jax.experimental.pallas import tpu as pltpu
from jax.experimental.pallas import tpu_sc as plsc

N_NODES = 10000
N_EDGES = 320000
D = 128

NC = 2
NS = 16
LANES = 16

CHUNK = 128
CHUNKS_PER_TILE = 80
GRP = 8
E_PAD = NC * NS * CHUNKS_PER_TILE * CHUNK
STRIPE = 624
TAIL = N_NODES - NS * STRIPE


def _matmul(x, W):
    def body(x_ref, w_ref, o_ref):
        o_ref[...] = jnp.dot(x_ref[...], w_ref[...],
                             preferred_element_type=jnp.float32)

    blk = 1000
    return pl.pallas_call(
        body,
        grid=(N_NODES // blk,),
        in_specs=[
            pl.BlockSpec((blk, D), lambda i: (i, 0)),
            pl.BlockSpec((D, D), lambda i: (0, 0)),
        ],
        out_specs=pl.BlockSpec((blk, D), lambda i: (i, 0)),
        out_shape=jax.ShapeDtypeStruct((N_NODES, D), jnp.float32),
    )(x, W)


def _combine(partials, b2):
    def body(p_ref, b_ref, o_ref):
        o_ref[...] = p_ref[0] + p_ref[1] + b_ref[...]

    blk = 1000
    return pl.pallas_call(
        body,
        grid=(N_NODES // blk,),
        in_specs=[
            pl.BlockSpec((2, blk, D), lambda i: (0, i, 0)),
            pl.BlockSpec((1, D), lambda i: (0, 0)),
        ],
        out_specs=pl.BlockSpec((blk, D), lambda i: (i, 0)),
        out_shape=jax.ShapeDtypeStruct((N_NODES, D), jnp.float32),
    )(partials, b2)


def _sc_spmm(src3, dst3, w3, support):
    mesh = plsc.VectorSubcoreMesh(core_axis_name="c", subcore_axis_name="s")

    @functools.partial(
        pl.kernel,
        out_type=jax.ShapeDtypeStruct((NC, N_NODES, D), jnp.float32),
        mesh=mesh,
        scratch_types=[
            pltpu.VMEM((GRP, CHUNK), jnp.int32),
            pltpu.VMEM((GRP, CHUNK), jnp.int32),
            pltpu.VMEM((GRP, CHUNK), jnp.float32),
            pltpu.VMEM((CHUNK, D), jnp.float32),
            pltpu.VMEM((CHUNK, D), jnp.float32),
            pltpu.VMEM_SHARED((N_NODES, D), jnp.float32),
            pltpu.SemaphoreType.DMA,
            pltpu.SemaphoreType.DMA,
        ],
    )
    def k(src_hbm, dst_hbm, w_hbm, sup_hbm, out_hbm,
          src_v, dst_v, w_v, rows_a, rows_b, acc, sem_a, sem_b):
        c = lax.axis_index("c")
        s = lax.axis_index("s")
        wid = c * NS + s
        base = wid * CHUNKS_PER_TILE

        def zbody(i, carry):
            for dd in range(D // LANES):
                rows_b[i, pl.ds(dd * LANES, LANES)] = jnp.zeros(
                    (LANES,), jnp.float32)
            return carry

        lax.fori_loop(0, CHUNK, zbody, 0)

        for z in range(4):
            pltpu.sync_copy(rows_b.at[pl.ds(0, CHUNK)],
                            acc.at[pl.ds(s * STRIPE + z * CHUNK, CHUNK)])
        pltpu.sync_copy(rows_b.at[pl.ds(0, STRIPE - 4 * CHUNK)],
                        acc.at[pl.ds(s * STRIPE + 4 * CHUNK,
                                     STRIPE - 4 * CHUNK)])

        @pl.when(s == NS - 1)
        def _():
            pltpu.sync_copy(rows_b.at[pl.ds(0, TAIL)],
                            acc.at[pl.ds(NS * STRIPE, TAIL)])

        plsc.subcore_barrier()

        def scale(buf, j):
            def mul_body(g, inner):
                i = g * LANES
                wv = w_v[j, pl.ds(i, LANES)]
                for e in range(LANES):
                    wsc = wv[e]
                    for dd in range(D // LANES):
                        sl = pl.ds(dd * LANES, LANES)
                        buf[i + e, sl] = buf[i + e, sl] * wsc
                return inner

            lax.fori_loop(0, CHUNK // LANES, mul_body, 0)

        def group_body(g, carry):
            gbase = base + g * GRP
            pltpu.sync_copy(src_hbm.at[pl.ds(gbase, GRP)], src_v)
            pltpu.sync_copy(dst_hbm.at[pl.ds(gbase, GRP)], dst_v)
            pltpu.sync_copy(w_hbm.at[pl.ds(gbase, GRP)], w_v)
            pltpu.async_copy(sup_hbm.at[src_v.at[0]], rows_a, sem_a)

            def pair_body(t, inner):
                j0 = 2 * t
                j1 = 2 * t + 1
                pltpu.async_copy(sup_hbm.at[src_v.at[j1]], rows_b, sem_b)
                pltpu.make_async_copy(sup_hbm.at[src_v.at[j0]], rows_a,
                                      sem_a).wait()
                scale(rows_a, j0)
                pltpu.sync_copy(rows_a, acc.at[dst_v.at[j0]], add=True)

                @pl.when(t < GRP // 2 - 1)
                def _():
                    pltpu.async_copy(sup_hbm.at[src_v.at[j0 + 2]], rows_a,
                                     sem_a)

                pltpu.make_async_copy(sup_hbm.at[src_v.at[j1]], rows_b,
                                      sem_b).wait()
                scale(rows_b, j1)
                pltpu.sync_copy(rows_b, acc.at[dst_v.at[j1]], add=True)
                return inner

            lax.fori_loop(0, GRP // 2, pair_body, 0)
            return carry

        lax.fori_loop(0, CHUNKS_PER_TILE // GRP, group_body, 0)
        plsc.subcore_barrier()

        pltpu.sync_copy(acc.at[pl.ds(s * STRIPE, STRIPE)],
                        out_hbm.at[c, pl.ds(s * STRIPE, STRIPE)])

        @pl.when(s == NS - 1)
        def _():
            pltpu.sync_copy(acc.at[pl.ds(NS * STRIPE, TAIL)],
                            out_hbm.at[c, pl.ds(NS * STRIPE, TAIL)])

    return k(src3, dst3, w3, support)


def kernel(x, edge_index, edge_weight, W, b):
    support = _matmul(x, W)

    src = edge_index[1].astype(jnp.int32)
    dst = edge_index[0].astype(jnp.int32)
    pad = E_PAD - N_EDGES
    zi = jnp.arange(pad, dtype=jnp.int32) % N_NODES
    src3 = jnp.concatenate([src, zi]).reshape(E_PAD // CHUNK, CHUNK)
    dst3 = jnp.concatenate([dst, zi]).reshape(E_PAD // CHUNK, CHUNK)
    w3 = jnp.concatenate(
        [edge_weight.astype(jnp.float32), jnp.zeros((pad,), jnp.float32)]
    ).reshape(E_PAD // CHUNK, CHUNK)

    partials = _sc_spmm(src3, dst3, w3, support)
    return _combine(partials, b.reshape(1, D))


# speedup vs baseline: 10.3214x; 1.0918x over previous
import functools

import jax
import jax.numpy as jnp
from jax import lax
from jax.experimental import pallas as pl
from jax.experimental.pallas import tpu as pltpu
from jax.experimental.pallas import tpu_sc as plsc

N_NODES = 10000
N_EDGES = 320000
D = 128

NC = 2
NS = 16
LANES = 16

CHUNK = 128
UNROLL = 12
CHUNKS_PER_TILE = 84
E_PAD = NC * NS * CHUNKS_PER_TILE * CHUNK
STRIPE = 624
TAIL = N_NODES - NS * STRIPE


def _matmul(x, W):
    def body(x_ref, w_ref, o_ref):
        o_ref[...] = jnp.dot(x_ref[...], w_ref[...],
                             preferred_element_type=jnp.float32)

    blk = 1000
    return pl.pallas_call(
        body,
        grid=(N_NODES // blk,),
        in_specs=[
            pl.BlockSpec((blk, D), lambda i: (i, 0)),
            pl.BlockSpec((D, D), lambda i: (0, 0)),
        ],
        out_specs=pl.BlockSpec((blk, D), lambda i: (i, 0)),
        out_shape=jax.ShapeDtypeStruct((N_NODES, D), jnp.float32),
    )(x, W)


def _combine(partials, b2):
    def body(p_ref, b_ref, o_ref):
        o_ref[...] = p_ref[0] + p_ref[1] + b_ref[...]

    blk = 1000
    return pl.pallas_call(
        body,
        grid=(N_NODES // blk,),
        in_specs=[
            pl.BlockSpec((2, blk, D), lambda i: (0, i, 0)),
            pl.BlockSpec((1, D), lambda i: (0, 0)),
        ],
        out_specs=pl.BlockSpec((blk, D), lambda i: (i, 0)),
        out_shape=jax.ShapeDtypeStruct((N_NODES, D), jnp.float32),
    )(partials, b2)


def _sc_spmm(src1, dst1, w1, support):
    mesh = plsc.VectorSubcoreMesh(core_axis_name="c", subcore_axis_name="s")

    @functools.partial(
        pl.kernel,
        out_type=jax.ShapeDtypeStruct((NC, N_NODES, D), jnp.float32),
        mesh=mesh,
        scratch_types=[
            pltpu.VMEM((4, CHUNK), jnp.int32),
            pltpu.VMEM((4, CHUNK), jnp.int32),
            pltpu.VMEM((3, CHUNK), jnp.float32),
            pltpu.VMEM((CHUNK, D), jnp.float32),
            pltpu.VMEM((CHUNK, D), jnp.float32),
            pltpu.VMEM((CHUNK, D), jnp.float32),
            pltpu.VMEM_SHARED((N_NODES, D), jnp.float32),
            pltpu.SemaphoreType.DMA,
            pltpu.SemaphoreType.DMA,
            pltpu.SemaphoreType.DMA,
            pltpu.SemaphoreType.DMA,
            pltpu.SemaphoreType.DMA,
            pltpu.SemaphoreType.DMA,
            pltpu.SemaphoreType.DMA,
            pltpu.SemaphoreType.DMA,
            pltpu.SemaphoreType.DMA,
        ],
    )
    def k(src_hbm, dst_hbm, w_hbm, sup_hbm, out_hbm,
          src_r, dst_r, w_r, b0, b1, b2, acc,
          g0, g1, g2, s0, s1, s2, i0, i1, i2):
        c = lax.axis_index("c")
        s = lax.axis_index("s")
        wid = c * NS + s
        ebase = wid * CHUNKS_PER_TILE * CHUNK
        bufs = (b0, b1, b2)
        gsem = (g0, g1, g2)
        ssem = (s0, s1, s2)
        isem = (i0, i1, i2)

        def zbody(i, carry):
            for dd in range(D // LANES):
                b2[i, pl.ds(dd * LANES, LANES)] = jnp.zeros(
                    (LANES,), jnp.float32)
            return carry

        lax.fori_loop(0, CHUNK, zbody, 0)

        for z in range(4):
            pltpu.sync_copy(b2.at[pl.ds(0, CHUNK)],
                            acc.at[pl.ds(s * STRIPE + z * CHUNK, CHUNK)])
        pltpu.sync_copy(b2.at[pl.ds(0, STRIPE - 4 * CHUNK)],
                        acc.at[pl.ds(s * STRIPE + 4 * CHUNK,
                                     STRIPE - 4 * CHUNK)])

        @pl.when(s == NS - 1)
        def _():
            pltpu.sync_copy(b2.at[pl.ds(0, TAIL)],
                            acc.at[pl.ds(NS * STRIPE, TAIL)])

        plsc.subcore_barrier()

        def idx_load(j, u):
            off = ebase + j * CHUNK
            pltpu.async_copy(src_hbm.at[pl.ds(off, CHUNK)],
                             src_r.at[u % 4], isem[u % 3])
            pltpu.async_copy(dst_hbm.at[pl.ds(off, CHUNK)],
                             dst_r.at[u % 4], isem[u % 3])
            pltpu.async_copy(w_hbm.at[pl.ds(off, CHUNK)],
                             w_r.at[u % 3], isem[u % 3])

        def wait_idx(u):
            pltpu.make_async_copy(src_hbm.at[pl.ds(0, CHUNK)],
                                  src_r.at[u % 4], isem[u % 3]).wait()
            pltpu.make_async_copy(dst_hbm.at[pl.ds(0, CHUNK)],
                                  dst_r.at[u % 4], isem[u % 3]).wait()
            pltpu.make_async_copy(w_hbm.at[pl.ds(0, CHUNK)],
                                  w_r.at[u % 3], isem[u % 3]).wait()

        def gather(u):
            pltpu.async_copy(sup_hbm.at[src_r.at[u % 4]], bufs[u % 3],
                             gsem[u % 3])

        def wait_gather(u):
            pltpu.make_async_copy(sup_hbm.at[src_r.at[u % 4]], bufs[u % 3],
                                  gsem[u % 3]).wait()

        def scatter(u):
            pltpu.async_copy(bufs[u % 3], acc.at[dst_r.at[u % 4]],
                             ssem[u % 3], add=True)

        def wait_scatter(u):
            pltpu.make_async_copy(bufs[u % 3], acc.at[dst_r.at[u % 4]],
                                  ssem[u % 3]).wait()

        def scale(u):
            buf = bufs[u % 3]

            def mul_body(g, inner):
                i = g * LANES
                wv = w_r[u % 3, pl.ds(i, LANES)]
                for e in range(LANES):
                    wsc = wv[e]
                    for dd in range(D // LANES):
                        sl = pl.ds(dd * LANES, LANES)
                        buf[i + e, sl] = buf[i + e, sl] * wsc
                return inner

            lax.fori_loop(0, CHUNK // LANES, mul_body, 0)

        T = CHUNKS_PER_TILE // UNROLL

        idx_load(0, 0)
        idx_load(1, 1)
        idx_load(2, 2)
        wait_idx(0)
        gather(0)

        def pipe_body(t, carry):
            j0 = t * UNROLL
            for u in range(UNROLL):
                j = j0 + u

                wait_gather(u)

                if u < 2:
                    @pl.when(t > 0)
                    def _(u=u):
                        wait_scatter(u - 2)
                else:
                    wait_scatter(u - 2)

                if u < UNROLL - 1:
                    wait_idx(u + 1)
                    gather(u + 1)
                else:
                    @pl.when(t < T - 1)
                    def _(u=u):
                        wait_idx(u + 1)
                        gather(u + 1)

                scale(u)
                scatter(u)

                if u < UNROLL - 3:
                    idx_load(j + 3, u + 3)
                else:
                    @pl.when(t < T - 1)
                    def _(u=u, j=j):
                        idx_load(j + 3, u + 3)
            return carry

        lax.fori_loop(0, T, pipe_body, 0)

        wait_scatter(UNROLL - 2)
        wait_scatter(UNROLL - 1)
        plsc.subcore_barrier()

        pltpu.sync_copy(acc.at[pl.ds(s * STRIPE, STRIPE)],
                        out_hbm.at[c, pl.ds(s * STRIPE, STRIPE)])

        @pl.when(s == NS - 1)
        def _():
            pltpu.sync_copy(acc.at[pl.ds(NS * STRIPE, TAIL)],
                            out_hbm.at[c, pl.ds(NS * STRIPE, TAIL)])

    return k(src1, dst1, w1, support)


def kernel(x, edge_index, edge_weight, W, b):
    support = _matmul(x, W)

    src = edge_index[1].astype(jnp.int32)
    dst = edge_index[0].astype(jnp.int32)
    pad = E_PAD - N_EDGES
    zi = jnp.arange(pad, dtype=jnp.int32) % N_NODES
    src1 = jnp.concatenate([src, zi])
    dst1 = jnp.concatenate([dst, zi])
    w1 = jnp.concatenate(
        [edge_weight.astype(jnp.float32), jnp.zeros((pad,), jnp.float32)])

    partials = _sc_spmm(src1, dst1, w1, support)
    return _combine(partials, b.reshape(1, D))
